# baseline (speedup 1.0000x reference)
import jax
import jax.numpy as jnp
from jax.experimental import pallas as pl

J = 64
BN = 256


def _gelu_exact(v):
    return 0.5 * v * (1.0 + jax.lax.erf(v * 0.7071067811865476))


def _build_m_kernel(src_ref, dst_ref, et_ref, m2_ref):
    e = src_ref.shape[0]
    k = jax.lax.broadcasted_iota(jnp.int32, (e, J), 1)
    s = (src_ref[...] == k).astype(jnp.float32)
    d = (dst_ref[...] == k).astype(jnp.float32)
    m1 = (et_ref[...] == 1).astype(jnp.float32)
    r = s - d
    rcat = jnp.concatenate([r * (1.0 - m1), r * m1], axis=1)
    m2_ref[...] = jax.lax.dot_general(
        d, rcat, (((0,), (0,)), ((), ())), preferred_element_type=jnp.float32)


def _main_kernel(x_ref, w0t_ref, w1t_ref, m2_ref, pose_ref, s_ref, b_ref,
                 o_ref):
    nj, bn, c = x_ref.shape
    xf = x_ref[...].reshape(nj * bn, c)
    xb = xf.astype(jnp.bfloat16)
    u0 = jnp.dot(xb, w0t_ref[...], preferred_element_type=jnp.float32)
    u1 = jnp.dot(xb, w1t_ref[...], preferred_element_type=jnp.float32)
    m2 = m2_ref[...]
    m0 = m2[:nj, :nj]
    m1 = m2[:nj, J:J + nj]
    u0r = u0.astype(jnp.bfloat16).reshape(nj, bn * c)
    u1r = u1.astype(jnp.bfloat16).reshape(nj, bn * c)
    y = (jnp.dot(m0.astype(jnp.bfloat16), u0r,
                 preferred_element_type=jnp.float32)
         + jnp.dot(m1.astype(jnp.bfloat16), u1r,
                   preferred_element_type=jnp.float32))
    agg = y.reshape(nj, bn, c) + pose_ref[...][:, None, :]
    h = x_ref[...] + _gelu_exact(agg)
    mean = jnp.mean(h, axis=-1, keepdims=True)
    msq = jnp.mean(h * h, axis=-1, keepdims=True)
    inv = jax.lax.rsqrt(msq - mean * mean + 1e-5)
    sc = inv * s_ref[...].reshape(1, 1, c)
    o_ref[...] = (h - mean) * sc + b_ref[...].reshape(1, 1, c)


def kernel(x, edge_index, edge_type, W0, W1, pose_table, ln_scale, ln_bias):
    n_b, n_j, c = x.shape
    e = edge_index.shape[1]
    src = edge_index[0].reshape(e, 1)
    dst = edge_index[1].reshape(e, 1)
    et = edge_type.reshape(e, 1)
    m2 = pl.pallas_call(
        _build_m_kernel,
        out_shape=jax.ShapeDtypeStruct((J, 2 * J), jnp.float32),
    )(src, dst, et)
    xt = jnp.transpose(x, (1, 0, 2))
    outt = pl.pallas_call(
        _main_kernel,
        grid=(n_b // BN,),
        in_specs=[
            pl.BlockSpec((n_j, BN, c), lambda i: (0, i, 0)),
            pl.BlockSpec((c, c), lambda i: (0, 0)),
            pl.BlockSpec((c, c), lambda i: (0, 0)),
            pl.BlockSpec((J, 2 * J), lambda i: (0, 0)),
            pl.BlockSpec((n_j, c), lambda i: (0, 0)),
            pl.BlockSpec((1, c), lambda i: (0, 0)),
            pl.BlockSpec((1, c), lambda i: (0, 0)),
        ],
        out_specs=pl.BlockSpec((n_j, BN, c), lambda i: (0, i, 0)),
        out_shape=jax.ShapeDtypeStruct((n_j, n_b, c), jnp.float32),
    )(xt, W0.T.astype(jnp.bfloat16), W1.T.astype(jnp.bfloat16), m2,
      pose_table, ln_scale.reshape(1, c), ln_bias.reshape(1, c))
    return jnp.transpose(outt, (1, 0, 2))


# speedup vs baseline: 1.0085x; 1.0085x over previous
import jax
import jax.numpy as jnp
from jax.experimental import pallas as pl

J = 64
BN = 128


def _gelu_exact(v):
    return 0.5 * v * (1.0 + jax.lax.erf(v * 0.7071067811865476))


def _build_m_kernel(src_ref, dst_ref, et_ref, m2_ref):
    e = src_ref.shape[0]
    k = jax.lax.broadcasted_iota(jnp.int32, (e, J), 1)
    s = (src_ref[...] == k).astype(jnp.float32)
    d = (dst_ref[...] == k).astype(jnp.float32)
    m1 = (et_ref[...] == 1).astype(jnp.float32)
    r = s - d
    rcat = jnp.concatenate([r * (1.0 - m1), r * m1], axis=1)
    m2_ref[...] = jax.lax.dot_general(
        d, rcat, (((0,), (0,)), ((), ())), preferred_element_type=jnp.float32)


def _main_kernel(x_ref, w0t_ref, w1t_ref, m2_ref, pose_ref, s_ref, b_ref,
                 o_ref):
    nj, bn, c = x_ref.shape
    xf = x_ref[...].reshape(nj * bn, c)
    xb = xf.astype(jnp.bfloat16)
    u0 = jnp.dot(xb, w0t_ref[...], preferred_element_type=jnp.float32)
    u1 = jnp.dot(xb, w1t_ref[...], preferred_element_type=jnp.float32)
    m2 = m2_ref[...]
    m0 = m2[:nj, :nj]
    m1 = m2[:nj, J:J + nj]
    u0r = u0.astype(jnp.bfloat16).reshape(nj, bn * c)
    u1r = u1.astype(jnp.bfloat16).reshape(nj, bn * c)
    y = (jnp.dot(m0.astype(jnp.bfloat16), u0r,
                 preferred_element_type=jnp.float32)
         + jnp.dot(m1.astype(jnp.bfloat16), u1r,
                   preferred_element_type=jnp.float32))
    agg = y.reshape(nj, bn, c) + pose_ref[...][:, None, :]
    h = x_ref[...] + _gelu_exact(agg)
    mean = jnp.mean(h, axis=-1, keepdims=True)
    msq = jnp.mean(h * h, axis=-1, keepdims=True)
    inv = jax.lax.rsqrt(msq - mean * mean + 1e-5)
    sc = inv * s_ref[...].reshape(1, 1, c)
    o_ref[...] = (h - mean) * sc + b_ref[...].reshape(1, 1, c)


def kernel(x, edge_index, edge_type, W0, W1, pose_table, ln_scale, ln_bias):
    n_b, n_j, c = x.shape
    e = edge_index.shape[1]
    src = edge_index[0].reshape(e, 1)
    dst = edge_index[1].reshape(e, 1)
    et = edge_type.reshape(e, 1)
    m2 = pl.pallas_call(
        _build_m_kernel,
        out_shape=jax.ShapeDtypeStruct((J, 2 * J), jnp.float32),
    )(src, dst, et)
    xt = jnp.transpose(x, (1, 0, 2))
    outt = pl.pallas_call(
        _main_kernel,
        grid=(n_b // BN,),
        in_specs=[
            pl.BlockSpec((n_j, BN, c), lambda i: (0, i, 0)),
            pl.BlockSpec((c, c), lambda i: (0, 0)),
            pl.BlockSpec((c, c), lambda i: (0, 0)),
            pl.BlockSpec((J, 2 * J), lambda i: (0, 0)),
            pl.BlockSpec((n_j, c), lambda i: (0, 0)),
            pl.BlockSpec((1, c), lambda i: (0, 0)),
            pl.BlockSpec((1, c), lambda i: (0, 0)),
        ],
        out_specs=pl.BlockSpec((n_j, BN, c), lambda i: (0, i, 0)),
        out_shape=jax.ShapeDtypeStruct((n_j, n_b, c), jnp.float32),
    )(xt, W0.T.astype(jnp.bfloat16), W1.T.astype(jnp.bfloat16), m2,
      pose_table, ln_scale.reshape(1, c), ln_bias.reshape(1, c))
    return jnp.transpose(outt, (1, 0, 2))
